# baseline (speedup 1.0000x reference)
import functools

import jax
import jax.numpy as jnp
from jax.experimental import pallas as pl
from jax.experimental.pallas import tpu as pltpu

_B, _N, _F, _H = 2, 512, 128, 64
_BI = 64


def _silu(x):
    return x * jax.nn.sigmoid(x)


def _egnn_kernel(nf_ref, pos_ref, valid_ref, adj_ref, Wemb_ref,
                 We1a0_ref, We1b0_ref, we1c0_ref, We2_0_ref, Wx1_0_ref, Wx2_0_ref,
                 Wh1a0_ref, Wh1b0_ref, Wh2_0_ref,
                 We1a1_ref, We1b1_ref, we1c1_ref, We2_1_ref,
                 Wh1a1_ref, Wh1b1_ref, Wh2_1_ref,
                 Wp1a_ref, Wp1b_ref, Wp2a_ref, Wp2b_ref,
                 out_ref,
                 h_s, pos_s, mask_s, A_s, C_s, agg_s, dp_s):
    f32 = jnp.float32
    vf = valid_ref[0].astype(f32)
    mask_s[...] = adj_ref[0].astype(f32) * vf * vf.reshape(_N, 1)
    h_s[...] = jnp.dot(nf_ref[0], Wemb_ref[...], preferred_element_type=f32)
    pos_s[...] = pos_ref[0]

    layer_ws = [
        (We1a0_ref, We1b0_ref, we1c0_ref, We2_0_ref, Wx1_0_ref, Wx2_0_ref,
         Wh1a0_ref, Wh1b0_ref, Wh2_0_ref),
        (We1a1_ref, We1b1_ref, we1c1_ref, We2_1_ref, None, None,
         Wh1a1_ref, Wh1b1_ref, Wh2_1_ref),
    ]

    for l, (We1a, We1b, we1c, We2, Wx1, Wx2, Wh1a, Wh1b, Wh2) in enumerate(layer_ws):
        h = h_s[...]
        pos = pos_s[...]
        A_s[...] = jnp.dot(h, We1a[...], preferred_element_type=f32)
        C = jnp.dot(h, We1b[...], preferred_element_type=f32)
        n2_row = jnp.sum(pos * pos, axis=1).reshape(1, _N)
        w1c = we1c[...].reshape(_H)

        def body(ib, carry, We2=We2, Wx1=Wx1, Wx2=Wx2, C=C,
                 n2_row=n2_row, w1c=w1c, l=l):
            sl = pl.ds(ib * _BI, _BI)
            posb = pos_s[sl, :]
            gram = jax.lax.dot_general(
                posb, pos_s[...], (((1,), (1,)), ((), ())),
                preferred_element_type=f32)
            n2b = jnp.sum(posb * posb, axis=1)
            d2 = n2b[:, None] + n2_row - 2.0 * gram
            pre = (A_s[sl, :][:, None, :] + C[None, :, :]
                   + d2[:, :, None] * w1c[None, None, :])
            s = _silu(pre).reshape(_BI * _N, _H)
            m = _silu(jnp.dot(s, We2[...], preferred_element_type=f32))
            m3 = m.reshape(_BI, _N, _H) * mask_s[sl, :][:, :, None]
            agg_s[sl, :] = jnp.sum(m3, axis=1)
            if l == 0:
                u = _silu(jnp.dot(m3.reshape(_BI * _N, _H), Wx1[...],
                                  preferred_element_type=f32))
                wx2 = Wx2[...].reshape(_H)
                coef = jnp.sum(u.reshape(_BI, _N, _H) * wx2[None, None, :],
                               axis=2)
                rw = jnp.sum(coef, axis=1)
                dp_s[sl, :] = (rw[:, None] * posb
                               - jnp.dot(coef, pos_s[...],
                                         preferred_element_type=f32)) / (_N - 1)
            return carry

        jax.lax.fori_loop(0, _N // _BI, body, 0)

        agg = agg_s[...]
        hid = _silu(jnp.dot(h, Wh1a[...], preferred_element_type=f32)
                    + jnp.dot(agg, Wh1b[...], preferred_element_type=f32))
        h_s[...] = h + jnp.dot(hid, Wh2[...], preferred_element_type=f32)
        if l == 0:
            pos_s[...] = pos + dp_s[...]

    h = h_s[...]
    p = jnp.dot(_silu(jnp.dot(h, Wp1a_ref[...], preferred_element_type=f32)),
                Wp1b_ref[...], preferred_element_type=f32)
    ps = jnp.sum(p, axis=0, keepdims=True)
    out = jnp.dot(_silu(jnp.dot(ps, Wp2a_ref[...], preferred_element_type=f32)),
                  Wp2b_ref[...], preferred_element_type=f32)
    out_ref[...] = out.reshape(1, 1, 1)


@functools.partial(jax.jit, static_argnames=("interpret",))
def _run(node_feat, pos, valid, adj, W_embed,
         We1_0, We2_0, Wx1_0, Wx2_0, Wh1_0, Wh2_0,
         We1_1, We2_1, Wx1_1, Wx2_1, Wh1_1, Wh2_1,
         Wp1a, Wp1b, Wp2a, Wp2b, interpret=False):
    H = _H
    args = (
        node_feat, pos, valid.reshape(_B, 1, _N), adj, W_embed,
        We1_0[:H], We1_0[H:2 * H], We1_0[2 * H:], We2_0, Wx1_0, Wx2_0,
        Wh1_0[:H], Wh1_0[H:], Wh2_0,
        We1_1[:H], We1_1[H:2 * H], We1_1[2 * H:], We2_1,
        Wh1_1[:H], Wh1_1[H:], Wh2_1,
        Wp1a, Wp1b, Wp2a, Wp2b,
    )
    batch_specs = [
        pl.BlockSpec((1, _N, _F), lambda b: (b, 0, 0)),
        pl.BlockSpec((1, _N, 3), lambda b: (b, 0, 0)),
        pl.BlockSpec((1, 1, _N), lambda b: (b, 0, 0)),
        pl.BlockSpec((1, _N, _N), lambda b: (b, 0, 0)),
    ]
    weight_specs = [pl.BlockSpec(a.shape, lambda b: (0,) * a.ndim)
                    for a in args[4:]]
    scratch = [
        pltpu.VMEM((_N, _H), jnp.float32),
        pltpu.VMEM((_N, 3), jnp.float32),
        pltpu.VMEM((_N, _N), jnp.float32),
        pltpu.VMEM((_N, _H), jnp.float32),
        pltpu.VMEM((_N, _H), jnp.float32),
        pltpu.VMEM((_N, _H), jnp.float32),
        pltpu.VMEM((_N, 3), jnp.float32),
    ]
    out = pl.pallas_call(
        _egnn_kernel,
        grid=(_B,),
        in_specs=batch_specs + weight_specs,
        out_specs=pl.BlockSpec((1, 1, 1), lambda b: (b, 0, 0)),
        out_shape=jax.ShapeDtypeStruct((_B, 1, 1), jnp.float32),
        scratch_shapes=scratch,
        interpret=interpret,
    )(*args)
    return out.reshape(_B, 1)


def kernel(node_feat, pos, valid, adj, W_embed,
           We1_0, We2_0, Wx1_0, Wx2_0, Wh1_0, Wh2_0,
           We1_1, We2_1, Wx1_1, Wx2_1, Wh1_1, Wh2_1,
           Wp1a, Wp1b, Wp2a, Wp2b):
    return _run(node_feat, pos, valid, adj, W_embed,
                We1_0, We2_0, Wx1_0, Wx2_0, Wh1_0, Wh2_0,
                We1_1, We2_1, Wx1_1, Wx2_1, Wh1_1, Wh2_1,
                Wp1a, Wp1b, Wp2a, Wp2b)
